# Initial kernel scaffold; baseline (speedup 1.0000x reference)
import jax
import jax.numpy as jnp
from jax.experimental import pallas as pl


def kernel(x, token_table, row_table, col_table):
    raise NotImplementedError("write your pallas kernel here")


# speedup vs baseline: 1.8724x; 1.8724x over previous
import functools

import jax
import jax.numpy as jnp
from jax import lax
from jax.experimental import pallas as pl
from jax.experimental.pallas import tpu as pltpu
from jax.experimental.pallas import tpu_sc as plsc


def _make_embed(n_rows, d, chunk, n_workers, nc):
    per_w = n_rows // n_workers
    n_chunks = per_w // chunk
    seq = 256

    @functools.partial(
        pl.kernel,
        out_type=jax.ShapeDtypeStruct((n_rows, d), jnp.float32),
        mesh=plsc.VectorSubcoreMesh(core_axis_name="c", subcore_axis_name="s"),
        scratch_types=[
            pltpu.VMEM((per_w // chunk, chunk), jnp.int32),
            pltpu.VMEM((seq, d), jnp.float32),
            pltpu.VMEM((chunk, d), jnp.float32),
            pltpu.SemaphoreType.DMA,
        ],
    )
    def embed(x2, pos, table, out, idx_v, pos_v, buf, sem):
        wid = lax.axis_index("s") * nc + lax.axis_index("c")
        base = wid * per_w
        pltpu.sync_copy(x2.at[pl.ds(wid * (per_w // d), per_w // d)], idx_v)
        pltpu.sync_copy(pos, pos_v)

        @functools.partial(lax.fori_loop, 0, n_chunks, init_val=None)
        def _chunks(c, _):
            pltpu.async_copy(table.at[idx_v.at[c]], buf, sem).wait()
            prow0 = (c % (seq // chunk)) * chunk

            @functools.partial(lax.fori_loop, 0, chunk, init_val=None)
            def _rows(i, _):
                for j in range(d // 16):
                    sl = pl.ds(j * 16, 16)
                    buf[i, sl] = buf[i, sl] + pos_v[prow0 + i, sl]
                return None

            pltpu.sync_copy(buf, out.at[pl.ds(base + c * chunk, chunk)])
            return None

        del _chunks

    return embed


def kernel(x, token_table, row_table, col_table):
    b = x.shape[0]
    x_flat = x.reshape(b, -1).astype(jnp.int32)
    seq_len = x_flat.shape[1]
    grid = row_table.shape[0]
    d = token_table.shape[1]
    pos = jnp.concatenate(
        [jnp.repeat(row_table, grid, axis=0), jnp.tile(col_table, (grid, 1))],
        axis=-1,
    )
    n_rows = b * seq_len
    x2 = x_flat.reshape(n_rows // d, d)
    info = plsc.get_sparse_core_info()
    nw = info.num_cores * info.num_subcores
    embed = _make_embed(n_rows, d, 128, nw, info.num_cores)
    out = embed(x2, pos, token_table)
    return out.reshape(b, seq_len, d)


# speedup vs baseline: 6.6060x; 3.5281x over previous
import functools

import jax
import jax.numpy as jnp
from jax import lax
from jax.experimental import pallas as pl
from jax.experimental.pallas import tpu as pltpu
from jax.experimental.pallas import tpu_sc as plsc


def _make_embed(n_rows, d, chunk, n_workers, nc):
    per_w = n_rows // n_workers
    n_chunks = per_w // chunk
    seq = 256
    blocks = seq // chunk

    @functools.partial(
        pl.kernel,
        out_type=jax.ShapeDtypeStruct((n_rows, d), jnp.float32),
        mesh=plsc.VectorSubcoreMesh(core_axis_name="c", subcore_axis_name="s"),
        scratch_types=[
            pltpu.VMEM((n_chunks, chunk), jnp.int32),
            pltpu.VMEM((seq, d), jnp.float32),
            pltpu.VMEM((2, chunk, d), jnp.float32),
            pltpu.VMEM((2, chunk, d), jnp.float32),
            pltpu.SemaphoreType.DMA,
            pltpu.SemaphoreType.DMA,
            pltpu.SemaphoreType.DMA,
            pltpu.SemaphoreType.DMA,
        ],
    )
    def embed(x2, pos, table, out, idx_v, pos_v, ibufs, obufs, sg0, sg1, so0, so1):
        wid = lax.axis_index("s") * nc + lax.axis_index("c")
        base = wid * per_w
        pltpu.sync_copy(x2.at[pl.ds(wid * (per_w // d), per_w // d)], idx_v)
        pltpu.sync_copy(pos, pos_v)

        sgs = (sg0, sg1)
        sos = (so0, so1)

        def gather(c, slot):
            return pltpu.make_async_copy(
                table.at[idx_v.at[c]], ibufs.at[slot], sgs[slot]
            )

        def put(c, slot):
            return pltpu.make_async_copy(
                obufs.at[slot], out.at[pl.ds(base + c * chunk, chunk)], sos[slot]
            )

        gather(0, 0).start()
        gather(1, 1).start()

        for c in range(n_chunks):
            slot = c % 2
            ibuf = ibufs.at[slot]
            obuf = obufs.at[slot]
            gather(c, slot).wait()
            if c >= 2:
                put(c - 2, slot).wait()
            prow0 = (c % blocks) * chunk

            @functools.partial(lax.fori_loop, 0, chunk, init_val=None)
            def _rows(i, _):
                for j in range(d // 16):
                    sl = pl.ds(j * 16, 16)
                    obuf[i, sl] = ibuf[i, sl] + pos_v[prow0 + i, sl]
                return None

            del _rows
            put(c, slot).start()
            if c + 2 < n_chunks:
                gather(c + 2, slot).start()

        put(n_chunks - 2, 0).wait()
        put(n_chunks - 1, 1).wait()

    return embed


def kernel(x, token_table, row_table, col_table):
    b = x.shape[0]
    x_flat = x.reshape(b, -1).astype(jnp.int32)
    seq_len = x_flat.shape[1]
    grid = row_table.shape[0]
    d = token_table.shape[1]
    pos = jnp.concatenate(
        [jnp.repeat(row_table, grid, axis=0), jnp.tile(col_table, (grid, 1))],
        axis=-1,
    )
    n_rows = b * seq_len
    x2 = x_flat.reshape(n_rows // d, d)
    info = plsc.get_sparse_core_info()
    nw = info.num_cores * info.num_subcores
    embed = _make_embed(n_rows, d, 128, nw, info.num_cores)
    out = embed(x2, pos, token_table)
    return out.reshape(b, seq_len, d)
